# Initial kernel scaffold; baseline (speedup 1.0000x reference)
import jax
import jax.numpy as jnp
from jax.experimental import pallas as pl


def kernel(x, W_r, b_r, W1, b1, W2, b2):
    raise NotImplementedError("write your pallas kernel here")


# speedup vs baseline: 3.1617x; 3.1617x over previous
import functools

import jax
import jax.numpy as jnp
from jax.experimental import pallas as pl
from jax.experimental.pallas import tpu as pltpu

DIM = 768
HID = 3072
E = 8
TOPK = 2
HT = 768
NH = HID // HT
SUB = 256
NEG_BIG = -1e30


def _router_body(x_ref, wr_ref, br_ref, logits_ref, mask_ref, gates_ref):
    x = x_ref[...]
    wr = wr_ref[...]
    logits = jax.lax.dot_general(
        x.astype(jnp.bfloat16), wr.astype(jnp.bfloat16),
        (((1,), (0,)), ((), ())),
        preferred_element_type=jnp.float32,
    ) + br_ref[...][None, :]
    logits_ref[...] = logits

    T = logits.shape[0]
    lane = jax.lax.broadcasted_iota(jnp.int32, (T, E), 1)
    m1 = jnp.max(logits, axis=1, keepdims=True)
    i1 = jnp.min(jnp.where(logits == m1, lane, E), axis=1, keepdims=True)
    masked = jnp.where(lane == i1, NEG_BIG, logits)
    m2 = jnp.max(masked, axis=1, keepdims=True)
    i2 = jnp.min(jnp.where(masked == m2, lane, E), axis=1, keepdims=True)

    a = jnp.exp(m2 - m1)
    w1 = 1.0 / (1.0 + a)
    w2 = a / (1.0 + a)

    sel1 = (lane == i1)
    sel2 = (lane == i2)
    mask_ref[...] = jnp.where(sel1 | sel2, 1.0, 0.0).astype(jnp.float32)
    gates_ref[...] = (jnp.where(sel1, w1, 0.0)
                      + jnp.where(sel2, w2, 0.0)).astype(jnp.float32)


def _ffn_body(x_ref, gates_ref, w1_ref, b1_ref, w2_ref, b2_ref, out_ref,
              acc_ref):
    e = pl.program_id(0)
    h = pl.program_id(1)
    T = x_ref.shape[0]

    lane = jax.lax.broadcasted_iota(jnp.int32, (T, E), 1)
    g = jnp.sum(jnp.where(lane == e, gates_ref[...], 0.0), axis=1,
                keepdims=True)

    w1 = w1_ref[0].astype(jnp.bfloat16)
    w2 = w2_ref[0].astype(jnp.bfloat16)
    b1v = b1_ref[0, 0, :][None, :]
    b2v = b2_ref[0, 0, :][None, :]

    @pl.when((e == 0) & (h == 0))
    def _init():
        acc_ref[...] = jnp.zeros_like(acc_ref)

    for s in range(T // SUB):
        sl = pl.ds(s * SUB, SUB)
        xs = x_ref[sl, :].astype(jnp.bfloat16)
        hcur = jax.lax.dot_general(
            xs, w1, (((1,), (0,)), ((), ())),
            preferred_element_type=jnp.float32) + b1v
        hcur = hcur * 0.5 * (1.0 + jax.lax.erf(hcur * 0.7071067811865476))
        part = jax.lax.dot_general(
            hcur.astype(jnp.bfloat16), w2, (((1,), (0,)), ((), ())),
            preferred_element_type=jnp.float32)
        gs = g[s * SUB:(s + 1) * SUB, :]
        contrib = gs * part
        acc_ref[sl, :] += contrib

    @pl.when(h == 0)
    def _bias2():
        acc_ref[...] += g * b2v

    @pl.when((e == E - 1) & (h == NH - 1))
    def _flush():
        out_ref[...] = acc_ref[...]


def kernel(x, W_r, b_r, W1, b1, W2, b2):
    batch, seq, dim = x.shape
    flat = x.reshape(-1, dim)
    T = flat.shape[0]

    logits, mask, gates = pl.pallas_call(
        _router_body,
        out_shape=[
            jax.ShapeDtypeStruct((T, E), jnp.float32),
            jax.ShapeDtypeStruct((T, E), jnp.float32),
            jax.ShapeDtypeStruct((T, E), jnp.float32),
        ],
    )(flat, W_r, b_r)

    b1r = b1.reshape(E, 1, HID)
    b2r = b2.reshape(E, 1, DIM)

    out_flat = pl.pallas_call(
        _ffn_body,
        grid=(E, NH),
        in_specs=[
            pl.BlockSpec((T, DIM), lambda e, h: (0, 0)),
            pl.BlockSpec((T, E), lambda e, h: (0, 0)),
            pl.BlockSpec((1, DIM, HT), lambda e, h: (e, 0, h)),
            pl.BlockSpec((1, 1, HT), lambda e, h: (e, 0, h)),
            pl.BlockSpec((1, HT, DIM), lambda e, h: (e, h, 0)),
            pl.BlockSpec((1, 1, DIM), lambda e, h: (e, 0, 0)),
        ],
        out_specs=pl.BlockSpec((T, DIM), lambda e, h: (0, 0)),
        out_shape=jax.ShapeDtypeStruct((T, DIM), jnp.float32),
        scratch_shapes=[pltpu.VMEM((T, DIM), jnp.float32)],
    )(flat, gates, W1, b1r, W2, b2r)

    return out_flat.reshape(batch, seq, dim), logits, mask


# speedup vs baseline: 3.4615x; 1.0948x over previous
import functools

import jax
import jax.numpy as jnp
from jax import lax
from jax.experimental import pallas as pl
from jax.experimental.pallas import tpu as pltpu
from jax.experimental.pallas import tpu_sc as plsc

DIM = 768
HID = 3072
E = 8
T = 2048
A = 2 * T
HT = 768
NH = HID // HT
SUB = 128
SEG = T // 2
NEG_BIG = -1e30

NC = 2
NS = 16
NW = NC * NS
APW = A // NW
TPW = T // NW
NCH = APW // 16
LANE16_SHIFTS = (1, 2, 4, 8)


def _dyn_gather(vec, idx):
    return lax.gather(
        vec, idx[:, None],
        dimension_numbers=lax.GatherDimensionNumbers(
            offset_dims=(), collapsed_slice_dims=(0,), start_index_map=(0,)),
        slice_sizes=(1,),
        mode=lax.GatherScatterMode.PROMISE_IN_BOUNDS)


def _prefix_sum(m, lane16):
    cs = m
    for sh in LANE16_SHIFTS:
        shifted = _dyn_gather(cs, jnp.maximum(lane16 - sh, 0))
        ge = jnp.minimum(jnp.maximum(lane16 - (sh - 1), 0), 1)
        cs = cs + ge * shifted
    return cs


def _splat_last(cs):
    return _dyn_gather(cs, jnp.full((16,), 15, jnp.int32))


def _router_body(x_ref, wr_ref, br_ref, logits_ref, mask_ref, eidx_ref,
                 wts_ref):
    x = x_ref[...]
    wr = wr_ref[...]
    logits = lax.dot_general(
        x.astype(jnp.bfloat16), wr.astype(jnp.bfloat16),
        (((1,), (0,)), ((), ())),
        preferred_element_type=jnp.float32,
    ) + br_ref[...][None, :]
    logits_ref[...] = logits

    lane = lax.broadcasted_iota(jnp.int32, (T, E), 1)
    m1 = jnp.max(logits, axis=1, keepdims=True)
    i1 = jnp.min(jnp.where(logits == m1, lane, E), axis=1, keepdims=True)
    masked = jnp.where(lane == i1, NEG_BIG, logits)
    m2 = jnp.max(masked, axis=1, keepdims=True)
    i2 = jnp.min(jnp.where(masked == m2, lane, E), axis=1, keepdims=True)

    a = jnp.exp(m2 - m1)
    w1 = 1.0 / (1.0 + a)
    w2 = a / (1.0 + a)

    sel1 = (lane == i1)
    sel2 = (lane == i2)
    mask_ref[...] = jnp.where(sel1 | sel2, 1.0, 0.0).astype(jnp.float32)
    eidx_ref[...] = jnp.concatenate([i1, i2], axis=1)
    wts_ref[...] = jnp.concatenate([w1, w2], axis=1)


def _dispatch_body(eflat, x_hbm, xs_out, slots_out, counts_out,
                   ev_v, tok_v, rank_v, slot_v, vec_v, hist_v, rows_v,
                   hist_sh, sem):
    cid = lax.axis_index("c")
    sid = lax.axis_index("s")
    wid = cid * NS + sid
    base_t = wid * TPW

    pltpu.sync_copy(eflat.at[pl.ds(base_t, TPW)], ev_v.at[pl.ds(0, TPW)])
    pltpu.sync_copy(eflat.at[pl.ds(T + base_t, TPW)],
                    ev_v.at[pl.ds(TPW, TPW)])

    lane16 = lax.broadcasted_iota(jnp.int32, (16,), 0)
    sid_vec = lane16 * 0 + sid
    for c in range(NCH):
        i16 = lane16 + c * 16
        tok_v[pl.ds(c * 16, 16)] = base_t + (i16 & (TPW - 1))

    run = [jnp.zeros((16,), jnp.int32) for _ in range(E)]
    for c in range(NCH):
        ev16 = ev_v[pl.ds(c * 16, 16)]
        rank16 = jnp.zeros((16,), jnp.int32)
        for e in range(E):
            m = jnp.where(ev16 == e, 1, 0)
            cs = _prefix_sum(m, lane16)
            rank16 = rank16 + m * (run[e] + cs - 1)
            run[e] = run[e] + _splat_last(cs)
        rank_v[pl.ds(c * 16, 16)] = rank16

    lh = jnp.zeros((16,), jnp.int32)
    for e in range(E):
        lh = lh + jnp.where(lane16 == e, 1, 0) * run[e]
    vec_v[...] = lh
    pltpu.sync_copy(vec_v, hist_sh.at[pl.ds(sid * 16, 16)])
    plsc.subcore_barrier()
    pltpu.sync_copy(hist_sh, hist_v)

    tb = jnp.zeros((16,), jnp.int32)
    cnt = jnp.zeros((16,), jnp.int32)
    for r in range(NS):
        row = hist_v[pl.ds(r * 16, 16)]
        lt = jnp.minimum(jnp.maximum(sid_vec - r, 0), 1)
        tb = tb + lt * row
        cnt = cnt + row

    @pl.when(sid == 0)
    def _write_counts():
        vec_v[...] = cnt
        pltpu.sync_copy(vec_v, counts_out.at[cid])

    for c in range(NCH):
        ev16 = ev_v[pl.ds(c * 16, 16)]
        tbg = _dyn_gather(tb, ev16)
        slot_v[pl.ds(c * 16, 16)] = (
            ev16 * T + cid * SEG + tbg + rank_v[pl.ds(c * 16, 16)])

    pltpu.sync_copy(slot_v.at[pl.ds(0, TPW)],
                    slots_out.at[pl.ds(base_t, TPW)])
    pltpu.sync_copy(slot_v.at[pl.ds(TPW, TPW)],
                    slots_out.at[pl.ds(T + base_t, TPW)])

    pltpu.async_copy(x_hbm.at[tok_v], rows_v, sem).wait()
    pltpu.async_copy(rows_v, xs_out.at[slot_v], sem).wait()


def _ffn_body(counts_ref, xs_ref, w1_ref, b1_ref, w2_ref, b2_ref, ys_ref,
              acc_ref):
    e = pl.program_id(0)
    h = pl.program_id(1)

    w1 = w1_ref[0].astype(jnp.bfloat16)
    w2 = w2_ref[0].astype(jnp.bfloat16)
    b1v = b1_ref[0, 0, :][None, :]
    b2v = b2_ref[0, 0, :][None, :]

    @pl.when(h == 0)
    def _init():
        acc_ref[...] = jnp.broadcast_to(b2v, acc_ref.shape)

    def seg_loop(seg_base, count):
        nsub = (count + SUB - 1) // SUB

        def body(s, _):
            off = seg_base + s * SUB
            xs = xs_ref[0, pl.ds(off, SUB), :].astype(jnp.bfloat16)
            hcur = lax.dot_general(
                xs, w1, (((1,), (0,)), ((), ())),
                preferred_element_type=jnp.float32) + b1v
            hcur = hcur * 0.5 * (1.0 + lax.erf(hcur * 0.7071067811865476))
            part = lax.dot_general(
                hcur.astype(jnp.bfloat16), w2, (((1,), (0,)), ((), ())),
                preferred_element_type=jnp.float32)
            acc_ref[pl.ds(off, SUB), :] += part
            return 0

        lax.fori_loop(0, nsub, body, 0)

    seg_loop(0, counts_ref[e])
    seg_loop(SEG, counts_ref[16 + e])

    @pl.when(h == NH - 1)
    def _flush():
        ys_ref[0] = acc_ref[...]


def _gatherpair_body(ys_hbm, slots_hbm, sel0_hbm, sel1_hbm, slot_v, rows_v,
                     sem):
    cid = lax.axis_index("c")
    sid = lax.axis_index("s")
    wid = cid * NS + sid
    base_t = wid * TPW

    pltpu.sync_copy(slots_hbm.at[pl.ds(base_t, TPW)], slot_v)
    pltpu.async_copy(ys_hbm.at[slot_v], rows_v, sem).wait()
    pltpu.sync_copy(rows_v, sel0_hbm.at[pl.ds(base_t, TPW)])

    pltpu.sync_copy(slots_hbm.at[pl.ds(T + base_t, TPW)], slot_v)
    pltpu.async_copy(ys_hbm.at[slot_v], rows_v, sem).wait()
    pltpu.sync_copy(rows_v, sel1_hbm.at[pl.ds(base_t, TPW)])


def _wsum_body(wts_ref, sel0_ref, sel1_ref, out_ref):
    w0 = wts_ref[:, 0:1]
    w1 = wts_ref[:, 1:2]
    out_ref[...] = w0 * sel0_ref[...] + w1 * sel1_ref[...]


def kernel(x, W_r, b_r, W1, b1, W2, b2):
    batch, seq, dim = x.shape
    flat = x.reshape(-1, dim)

    logits, mask, eidx, wts = pl.pallas_call(
        _router_body,
        out_shape=[
            jax.ShapeDtypeStruct((T, E), jnp.float32),
            jax.ShapeDtypeStruct((T, E), jnp.float32),
            jax.ShapeDtypeStruct((T, 2), jnp.int32),
            jax.ShapeDtypeStruct((T, 2), jnp.float32),
        ],
    )(flat, W_r, b_r)

    eflat = jnp.concatenate([eidx[:, 0], eidx[:, 1]])

    mesh = plsc.VectorSubcoreMesh(core_axis_name="c", subcore_axis_name="s")
    xs_flat, slots, counts = pl.kernel(
        _dispatch_body,
        out_type=[
            jax.ShapeDtypeStruct((E * T, DIM), jnp.float32),
            jax.ShapeDtypeStruct((A,), jnp.int32),
            jax.ShapeDtypeStruct((NC, 16), jnp.int32),
        ],
        mesh=mesh,
        scratch_types=[
            pltpu.VMEM((APW,), jnp.int32),
            pltpu.VMEM((APW,), jnp.int32),
            pltpu.VMEM((APW,), jnp.int32),
            pltpu.VMEM((APW,), jnp.int32),
            pltpu.VMEM((16,), jnp.int32),
            pltpu.VMEM((NS * 16,), jnp.int32),
            pltpu.VMEM((APW, DIM), jnp.float32),
            pltpu.VMEM_SHARED((NS * 16,), jnp.int32),
            pltpu.SemaphoreType.DMA,
        ],
    )(eflat, flat)

    xs = xs_flat.reshape(E, T, DIM)
    counts_flat = counts.reshape(NC * 16)

    b1r = b1.reshape(E, 1, HID)
    b2r = b2.reshape(E, 1, DIM)

    ys = pl.pallas_call(
        _ffn_body,
        grid_spec=pltpu.PrefetchScalarGridSpec(
            num_scalar_prefetch=1,
            grid=(E, NH),
            in_specs=[
                pl.BlockSpec((1, T, DIM), lambda e, h, c: (e, 0, 0)),
                pl.BlockSpec((1, DIM, HT), lambda e, h, c: (e, 0, h)),
                pl.BlockSpec((1, 1, HT), lambda e, h, c: (e, 0, h)),
                pl.BlockSpec((1, HT, DIM), lambda e, h, c: (e, h, 0)),
                pl.BlockSpec((1, 1, DIM), lambda e, h, c: (e, 0, 0)),
            ],
            out_specs=pl.BlockSpec((1, T, DIM), lambda e, h, c: (e, 0, 0)),
            scratch_shapes=[pltpu.VMEM((T, DIM), jnp.float32)],
        ),
        out_shape=jax.ShapeDtypeStruct((E, T, DIM), jnp.float32),
    )(counts_flat, xs, W1, b1r, W2, b2r)

    sel0, sel1 = pl.kernel(
        _gatherpair_body,
        out_type=[
            jax.ShapeDtypeStruct((T, DIM), jnp.float32),
            jax.ShapeDtypeStruct((T, DIM), jnp.float32),
        ],
        mesh=plsc.VectorSubcoreMesh(core_axis_name="c",
                                    subcore_axis_name="s"),
        scratch_types=[
            pltpu.VMEM((TPW,), jnp.int32),
            pltpu.VMEM((TPW, DIM), jnp.float32),
            pltpu.SemaphoreType.DMA,
        ],
    )(ys.reshape(E * T, DIM), slots)

    out_flat = pl.pallas_call(
        _wsum_body,
        out_shape=jax.ShapeDtypeStruct((T, DIM), jnp.float32),
    )(wts, sel0, sel1)

    return out_flat.reshape(batch, seq, dim), logits, mask
